# baseline (speedup 1.0000x reference)
import functools

import jax
import jax.numpy as jnp
from jax.experimental import pallas as pl

_B = 16
_N1, _M1, _NS1, _R1 = 1024, 512, 32, 0.2
_N2, _M2, _NS2, _R2 = 512, 128, 64, 0.4


def _lrelu(x):
    return jnp.where(x >= 0, x, 0.01 * x)


def _fps_body(x, y, z, n, npoint, cx_ref, cy_ref, cz_ref):
    iota = jax.lax.broadcasted_iota(jnp.int32, (_B, n), 1)
    iota_c = jax.lax.broadcasted_iota(jnp.int32, (_B, npoint), 1)

    def body(i, carry):
        dist, far, cx, cy, cz = carry
        oh = iota == far
        xf = jnp.sum(jnp.where(oh, x, 0.0), axis=1, keepdims=True)
        yf = jnp.sum(jnp.where(oh, y, 0.0), axis=1, keepdims=True)
        zf = jnp.sum(jnp.where(oh, z, 0.0), axis=1, keepdims=True)
        sel = iota_c == i
        cx = jnp.where(sel, xf, cx)
        cy = jnp.where(sel, yf, cy)
        cz = jnp.where(sel, zf, cz)
        dx = x - xf
        dy = y - yf
        dz = z - zf
        d = dx * dx + dy * dy + dz * dz
        dist = jnp.minimum(dist, d)
        m = jnp.max(dist, axis=1, keepdims=True)
        far = jnp.min(jnp.where(dist == m, iota, n), axis=1, keepdims=True)
        return dist, far, cx, cy, cz

    dist0 = jnp.full((_B, n), 1e10, dtype=jnp.float32)
    far0 = jnp.zeros((_B, 1), dtype=jnp.int32)
    c0 = jnp.zeros((_B, npoint), dtype=jnp.float32)
    _, _, cx, cy, cz = jax.lax.fori_loop(0, npoint, body,
                                         (dist0, far0, c0, c0, c0))
    cx_ref[:, :] = cx
    cy_ref[:, :] = cy
    cz_ref[:, :] = cz


def _fps_kernel(xt_ref, c1x_ref, c1y_ref, c1z_ref, c2x_ref, c2y_ref, c2z_ref):
    x = xt_ref[0]
    y = xt_ref[1]
    z = xt_ref[2]
    _fps_body(x, y, z, _N1, _M1, c1x_ref, c1y_ref, c1z_ref)
    x2 = c1x_ref[:, :]
    y2 = c1y_ref[:, :]
    z2 = c1z_ref[:, :]
    _fps_body(x2, y2, z2, _M1, _M2, c2x_ref, c2y_ref, c2z_ref)


def _run_fps(xt):
    f32 = jnp.float32
    outs = [
        jax.ShapeDtypeStruct((_B, _M1), f32), jax.ShapeDtypeStruct((_B, _M1), f32),
        jax.ShapeDtypeStruct((_B, _M1), f32), jax.ShapeDtypeStruct((_B, _M2), f32),
        jax.ShapeDtypeStruct((_B, _M2), f32), jax.ShapeDtypeStruct((_B, _M2), f32),
    ]
    return pl.pallas_call(_fps_kernel, out_shape=outs)(xt)


def _bq_kernel(pt_ref, ct_ref, idxt_ref, *, n, m, ns, r2):
    px = pt_ref[0, 0:1, :]
    py = pt_ref[0, 1:2, :]
    pz = pt_ref[0, 2:3, :]
    cx = ct_ref[0, 0:1, :].T
    cy = ct_ref[0, 1:2, :].T
    cz = ct_ref[0, 2:3, :].T
    dx = cx - px
    dy = cy - py
    dz = cz - pz
    d2 = dx * dx + dy * dy + dz * dz
    big = float(n)
    iota = jax.lax.broadcasted_iota(jnp.int32, (m, n), 1).astype(jnp.float32)
    v0 = jnp.where(d2 <= r2, iota, big)
    iota_s = jax.lax.broadcasted_iota(jnp.int32, (ns, m), 0)

    def body(k, carry):
        v, idx0, out = carry
        mn = jnp.min(v, axis=1, keepdims=True)
        sel = jnp.where(mn >= big, idx0, mn)
        idx0 = jnp.where(k == 0, sel, idx0)
        out = jnp.where(iota_s == k, sel.T, out)
        v = jnp.where(v == mn, big, v)
        return v, idx0, out

    _, _, out = jax.lax.fori_loop(
        0, ns, body,
        (v0, jnp.zeros((m, 1), jnp.float32), jnp.zeros((ns, m), jnp.float32)))
    idxt_ref[0] = out.astype(jnp.int32)


def _run_bq(pt, ct, n, m, ns, r):
    kfn = functools.partial(_bq_kernel, n=n, m=m, ns=ns, r2=r * r)
    return pl.pallas_call(
        kfn,
        grid=(_B,),
        in_specs=[
            pl.BlockSpec((1, 3, n), lambda b: (b, 0, 0)),
            pl.BlockSpec((1, 3, m), lambda b: (b, 0, 0)),
        ],
        out_specs=pl.BlockSpec((1, ns, m), lambda b: (b, 0, 0)),
        out_shape=jax.ShapeDtypeStruct((_B, ns, m), jnp.int32),
    )(pt, ct)


def _sa_kernel(pt_ref, ct_ref, idxt_ref, w0t_ref, b0_ref, w1t_ref, b1_ref,
               w2t_ref, b2_ref, out_ref, *, m, ns, nch):
    w0t = w0t_ref[:, :]
    a_t = jnp.dot(w0t, pt_ref[0], preferred_element_type=jnp.float32)
    bc = jnp.dot(w0t[:, 0:3], ct_ref[0],
                 preferred_element_type=jnp.float32)
    c1 = w0t.shape[0]
    c2 = w2t_ref.shape[0]
    off = bc - b0_ref[:, :]
    w1t = w1t_ref[:, :]
    b1 = b1_ref[:, :]
    w2t = w2t_ref[:, :]
    b2 = b2_ref[:, :]

    def body(k, acc):
        idxk = idxt_ref[0, pl.ds(k, 1), :]
        idxb = jnp.broadcast_to(idxk, (c1, m))
        lo = jax.lax.rem(idxb, 128)
        hi = jax.lax.div(idxb, 128)
        g = jnp.zeros((c1, m), jnp.float32)
        for ch in range(nch):
            src = a_t[:, ch * 128:(ch + 1) * 128]
            gc = jnp.take_along_axis(src, lo, axis=1,
                                     mode="promise_in_bounds")
            g = jnp.where(hi == ch, gc, g)
        h = _lrelu(g - off)
        h = _lrelu(jnp.dot(w1t, h, preferred_element_type=jnp.float32) + b1)
        h = _lrelu(jnp.dot(w2t, h, preferred_element_type=jnp.float32) + b2)
        return jnp.maximum(acc, h)

    out_ref[0] = jax.lax.fori_loop(
        0, ns, body, jnp.full((c2, m), -jnp.inf, jnp.float32))


def _run_sa(pt, ct, idxt, w0t, b0, w1t, b1, w2t, b2, n, m, ns):
    c1, cin = w0t.shape
    c2 = w2t.shape[0]
    kfn = functools.partial(_sa_kernel, m=m, ns=ns, nch=n // 128)
    wspec = lambda s: pl.BlockSpec(s, lambda b: (0,) * len(s))
    return pl.pallas_call(
        kfn,
        grid=(_B,),
        in_specs=[
            pl.BlockSpec((1, cin, n), lambda b: (b, 0, 0)),
            pl.BlockSpec((1, 3, m), lambda b: (b, 0, 0)),
            pl.BlockSpec((1, ns, m), lambda b: (b, 0, 0)),
            wspec((c1, cin)), wspec((c1, 1)),
            wspec((c1, c1)), wspec((c1, 1)),
            wspec((c2, c1)), wspec((c2, 1)),
        ],
        out_specs=pl.BlockSpec((1, c2, m), lambda b: (b, 0, 0)),
        out_shape=jax.ShapeDtypeStruct((_B, c2, m), jnp.float32),
    )(pt, ct, idxt, w0t, b0, w1t, b1, w2t, b2)


def _sa3_kernel(ft_ref, w0t_ref, b0_ref, w1t_ref, b1_ref, w2t_ref, b2_ref,
                out_ref):
    h = _lrelu(jnp.dot(w0t_ref[:, :], ft_ref[0],
                       preferred_element_type=jnp.float32) + b0_ref[:, :])
    h = _lrelu(jnp.dot(w1t_ref[:, :], h,
                       preferred_element_type=jnp.float32) + b1_ref[:, :])
    h = _lrelu(jnp.dot(w2t_ref[:, :], h,
                       preferred_element_type=jnp.float32) + b2_ref[:, :])
    out_ref[0] = jnp.max(h, axis=1, keepdims=True).T


def _run_sa3(ft, w0t, b0, w1t, b1, w2t, b2):
    cin = ft.shape[1]
    c0, c1, c2 = w0t.shape[0], w1t.shape[0], w2t.shape[0]
    wspec = lambda s: pl.BlockSpec(s, lambda b: (0,) * len(s))
    return pl.pallas_call(
        _sa3_kernel,
        grid=(_B,),
        in_specs=[
            pl.BlockSpec((1, cin, _M2), lambda b: (b, 0, 0)),
            wspec((c0, cin)), wspec((c0, 1)),
            wspec((c1, c0)), wspec((c1, 1)),
            wspec((c2, c1)), wspec((c2, 1)),
        ],
        out_specs=pl.BlockSpec((1, 1, c2), lambda b: (b, 0, 0)),
        out_shape=jax.ShapeDtypeStruct((_B, 1, c2), jnp.float32),
    )(ft, w0t, b0, w1t, b1, w2t, b2).reshape(_B, c2)


def _head_kernel(f_ref, w1_ref, b1_ref, w2_ref, b2_ref, w3_ref, b3_ref,
                 out_ref):
    h = _lrelu(jnp.dot(f_ref[:, :], w1_ref[:, :],
                       preferred_element_type=jnp.float32) + b1_ref[:, :])
    h = _lrelu(jnp.dot(h, w2_ref[:, :],
                       preferred_element_type=jnp.float32) + b2_ref[:, :])
    l = jnp.dot(h, w3_ref[:, :], preferred_element_type=jnp.float32) \
        + b3_ref[:, :]
    mx = jnp.max(l, axis=1, keepdims=True)
    e = jnp.exp(l - mx)
    out_ref[:, :] = e / jnp.sum(e, axis=1, keepdims=True)


def _run_head(f, w1, b1, w2, b2, w3, b3):
    nc = w3.shape[1]
    return pl.pallas_call(
        _head_kernel,
        out_shape=jax.ShapeDtypeStruct((_B, nc), jnp.float32),
    )(f, w1, b1, w2, b2, w3, b3)


def kernel(input, sa1_w0, sa1_b0, sa1_w1, sa1_b1, sa1_w2, sa1_b2,
           sa2_w0, sa2_b0, sa2_w1, sa2_b1, sa2_w2, sa2_b2,
           sa3_w0, sa3_b0, sa3_w1, sa3_b1, sa3_w2, sa3_b2,
           d1_w, d1_b, d2_w, d2_b, d3_w, d3_b):
    col = lambda b: b.reshape(-1, 1)
    row = lambda b: b.reshape(1, -1)
    xt = input.transpose(2, 0, 1)
    xbt = input.transpose(0, 2, 1)
    c1x, c1y, c1z, c2x, c2y, c2z = _run_fps(xt)
    ct1 = jnp.stack([c1x, c1y, c1z], axis=1)
    ct2 = jnp.stack([c2x, c2y, c2z], axis=1)

    idxt1 = _run_bq(xbt, ct1, _N1, _M1, _NS1, _R1)
    pts1t = _run_sa(xbt, ct1, idxt1, sa1_w0.T, col(sa1_b0), sa1_w1.T,
                    col(sa1_b1), sa1_w2.T, col(sa1_b2),
                    _N1, _M1, _NS1)

    p2t = jnp.concatenate([ct1, pts1t], axis=1)
    idxt2 = _run_bq(ct1, ct2, _M1, _M2, _NS2, _R2)
    pts2t = _run_sa(p2t, ct2, idxt2, sa2_w0.T, col(sa2_b0), sa2_w1.T,
                    col(sa2_b1), sa2_w2.T, col(sa2_b2),
                    _M1, _M2, _NS2)

    f3t = jnp.concatenate([ct2, pts2t], axis=1)
    feat = _run_sa3(f3t, sa3_w0.T, col(sa3_b0), sa3_w1.T, col(sa3_b1),
                    sa3_w2.T, col(sa3_b2))
    return _run_head(feat, d1_w, row(d1_b), d2_w, row(d2_b),
                     d3_w, row(d3_b))


# speedup vs baseline: 1.3072x; 1.3072x over previous
import functools

import jax
import jax.numpy as jnp
from jax.experimental import pallas as pl

_B = 16
_N1, _M1, _NS1, _R1 = 1024, 512, 32, 0.2
_N2, _M2, _NS2, _R2 = 512, 128, 64, 0.4


def _lrelu(x):
    return jnp.where(x >= 0, x, 0.01 * x)


def _fps_body(x, y, z, n, npoint, cx_ref, cy_ref, cz_ref):
    iota = jax.lax.broadcasted_iota(jnp.int32, (_B, n), 1)
    iota_c = jax.lax.broadcasted_iota(jnp.int32, (_B, npoint), 1)

    def body(i, carry):
        dist, far, cx, cy, cz = carry
        oh = iota == far
        xf = jnp.sum(jnp.where(oh, x, 0.0), axis=1, keepdims=True)
        yf = jnp.sum(jnp.where(oh, y, 0.0), axis=1, keepdims=True)
        zf = jnp.sum(jnp.where(oh, z, 0.0), axis=1, keepdims=True)
        sel = iota_c == i
        cx = jnp.where(sel, xf, cx)
        cy = jnp.where(sel, yf, cy)
        cz = jnp.where(sel, zf, cz)
        dx = x - xf
        dy = y - yf
        dz = z - zf
        d = dx * dx + dy * dy + dz * dz
        dist = jnp.minimum(dist, d)
        m = jnp.max(dist, axis=1, keepdims=True)
        far = jnp.min(jnp.where(dist == m, iota, n), axis=1, keepdims=True)
        return dist, far, cx, cy, cz

    dist0 = jnp.full((_B, n), 1e10, dtype=jnp.float32)
    far0 = jnp.zeros((_B, 1), dtype=jnp.int32)
    c0 = jnp.zeros((_B, npoint), dtype=jnp.float32)
    _, _, cx, cy, cz = jax.lax.fori_loop(0, npoint, body,
                                         (dist0, far0, c0, c0, c0))
    cx_ref[:, :] = cx
    cy_ref[:, :] = cy
    cz_ref[:, :] = cz


def _fps_kernel(xt_ref, c1x_ref, c1y_ref, c1z_ref, c2x_ref, c2y_ref, c2z_ref):
    x = xt_ref[0]
    y = xt_ref[1]
    z = xt_ref[2]
    _fps_body(x, y, z, _N1, _M1, c1x_ref, c1y_ref, c1z_ref)
    x2 = c1x_ref[:, :]
    y2 = c1y_ref[:, :]
    z2 = c1z_ref[:, :]
    _fps_body(x2, y2, z2, _M1, _M2, c2x_ref, c2y_ref, c2z_ref)


def _run_fps(xt):
    f32 = jnp.float32
    outs = [
        jax.ShapeDtypeStruct((_B, _M1), f32), jax.ShapeDtypeStruct((_B, _M1), f32),
        jax.ShapeDtypeStruct((_B, _M1), f32), jax.ShapeDtypeStruct((_B, _M2), f32),
        jax.ShapeDtypeStruct((_B, _M2), f32), jax.ShapeDtypeStruct((_B, _M2), f32),
    ]
    return pl.pallas_call(_fps_kernel, out_shape=outs)(xt)


def _bq_kernel(pt_ref, ct_ref, idxt_ref, *, n, m, ns, r2):
    px = pt_ref[0, 0:1, :]
    py = pt_ref[0, 1:2, :]
    pz = pt_ref[0, 2:3, :]
    cx = ct_ref[0, 0:1, :].T
    cy = ct_ref[0, 1:2, :].T
    cz = ct_ref[0, 2:3, :].T
    dx = cx - px
    dy = cy - py
    dz = cz - pz
    d2 = dx * dx + dy * dy + dz * dz
    big = float(n)
    iota = jax.lax.broadcasted_iota(jnp.int32, (m, n), 1).astype(jnp.float32)
    v0 = jnp.where(d2 <= r2, iota, big)
    iota_s = jax.lax.broadcasted_iota(jnp.int32, (ns, m), 0)

    def body(k, carry):
        v, idx0, out = carry
        mn = jnp.min(v, axis=1, keepdims=True)
        sel = jnp.where(mn >= big, idx0, mn)
        idx0 = jnp.where(k == 0, sel, idx0)
        out = jnp.where(iota_s == k, sel.T, out)
        v = jnp.where(v == mn, big, v)
        return v, idx0, out

    _, _, out = jax.lax.fori_loop(
        0, ns, body,
        (v0, jnp.zeros((m, 1), jnp.float32), jnp.zeros((ns, m), jnp.float32)))
    idxt_ref[0] = out.astype(jnp.int32)


def _run_bq(pt, ct, n, m, ns, r):
    kfn = functools.partial(_bq_kernel, n=n, m=m, ns=ns, r2=r * r)
    return pl.pallas_call(
        kfn,
        grid=(_B,),
        in_specs=[
            pl.BlockSpec((1, 3, n), lambda b: (b, 0, 0)),
            pl.BlockSpec((1, 3, m), lambda b: (b, 0, 0)),
        ],
        out_specs=pl.BlockSpec((1, ns, m), lambda b: (b, 0, 0)),
        out_shape=jax.ShapeDtypeStruct((_B, ns, m), jnp.int32),
    )(pt, ct)


def _sa_kernel(pt_ref, ct_ref, idxt_ref, w0t_ref, b0_ref, w1t_ref, b1_ref,
               w2t_ref, b2_ref, out_ref, *, m, ns, nch, g_blk):
    w0t = w0t_ref[:, :]
    a_t = jnp.dot(w0t, pt_ref[0], preferred_element_type=jnp.float32)
    bc = jnp.dot(w0t[:, 0:3], ct_ref[0],
                 preferred_element_type=jnp.float32)
    c1 = w0t.shape[0]
    c2 = w2t_ref.shape[0]
    off = bc - b0_ref[:, :]
    w1t = w1t_ref[:, :]
    b1 = b1_ref[:, :]
    w2t = w2t_ref[:, :]
    b2 = b2_ref[:, :]

    def body(k, acc):
        parts = []
        for t in range(g_blk):
            idxk = idxt_ref[0, pl.ds(k * g_blk + t, 1), :]
            idxb = jnp.broadcast_to(idxk, (c1, m))
            lo = jax.lax.rem(idxb, 128)
            hi = jax.lax.div(idxb, 128)
            g = jnp.zeros((c1, m), jnp.float32)
            for ch in range(nch):
                src = a_t[:, ch * 128:(ch + 1) * 128]
                gc = jnp.take_along_axis(src, lo, axis=1,
                                         mode="promise_in_bounds")
                g = jnp.where(hi == ch, gc, g)
            parts.append(_lrelu(g - off))
        h = jnp.concatenate(parts, axis=1)
        h = _lrelu(jnp.dot(w1t, h, preferred_element_type=jnp.float32) + b1)
        h = _lrelu(jnp.dot(w2t, h, preferred_element_type=jnp.float32) + b2)
        for t in range(g_blk):
            acc = jnp.maximum(acc, h[:, t * m:(t + 1) * m])
        return acc

    out_ref[0] = jax.lax.fori_loop(
        0, ns // g_blk, body, jnp.full((c2, m), -jnp.inf, jnp.float32))


def _run_sa(pt, ct, idxt, w0t, b0, w1t, b1, w2t, b2, n, m, ns):
    c1, cin = w0t.shape
    c2 = w2t.shape[0]
    kfn = functools.partial(_sa_kernel, m=m, ns=ns, nch=n // 128, g_blk=8)
    wspec = lambda s: pl.BlockSpec(s, lambda b: (0,) * len(s))
    return pl.pallas_call(
        kfn,
        grid=(_B,),
        in_specs=[
            pl.BlockSpec((1, cin, n), lambda b: (b, 0, 0)),
            pl.BlockSpec((1, 3, m), lambda b: (b, 0, 0)),
            pl.BlockSpec((1, ns, m), lambda b: (b, 0, 0)),
            wspec((c1, cin)), wspec((c1, 1)),
            wspec((c1, c1)), wspec((c1, 1)),
            wspec((c2, c1)), wspec((c2, 1)),
        ],
        out_specs=pl.BlockSpec((1, c2, m), lambda b: (b, 0, 0)),
        out_shape=jax.ShapeDtypeStruct((_B, c2, m), jnp.float32),
    )(pt, ct, idxt, w0t, b0, w1t, b1, w2t, b2)


def _sa3_kernel(ft_ref, w0t_ref, b0_ref, w1t_ref, b1_ref, w2t_ref, b2_ref,
                out_ref):
    h = _lrelu(jnp.dot(w0t_ref[:, :], ft_ref[0],
                       preferred_element_type=jnp.float32) + b0_ref[:, :])
    h = _lrelu(jnp.dot(w1t_ref[:, :], h,
                       preferred_element_type=jnp.float32) + b1_ref[:, :])
    h = _lrelu(jnp.dot(w2t_ref[:, :], h,
                       preferred_element_type=jnp.float32) + b2_ref[:, :])
    out_ref[0] = jnp.max(h, axis=1, keepdims=True).T


def _run_sa3(ft, w0t, b0, w1t, b1, w2t, b2):
    cin = ft.shape[1]
    c0, c1, c2 = w0t.shape[0], w1t.shape[0], w2t.shape[0]
    wspec = lambda s: pl.BlockSpec(s, lambda b: (0,) * len(s))
    return pl.pallas_call(
        _sa3_kernel,
        grid=(_B,),
        in_specs=[
            pl.BlockSpec((1, cin, _M2), lambda b: (b, 0, 0)),
            wspec((c0, cin)), wspec((c0, 1)),
            wspec((c1, c0)), wspec((c1, 1)),
            wspec((c2, c1)), wspec((c2, 1)),
        ],
        out_specs=pl.BlockSpec((1, 1, c2), lambda b: (b, 0, 0)),
        out_shape=jax.ShapeDtypeStruct((_B, 1, c2), jnp.float32),
    )(ft, w0t, b0, w1t, b1, w2t, b2).reshape(_B, c2)


def _head_kernel(f_ref, w1_ref, b1_ref, w2_ref, b2_ref, w3_ref, b3_ref,
                 out_ref):
    h = _lrelu(jnp.dot(f_ref[:, :], w1_ref[:, :],
                       preferred_element_type=jnp.float32) + b1_ref[:, :])
    h = _lrelu(jnp.dot(h, w2_ref[:, :],
                       preferred_element_type=jnp.float32) + b2_ref[:, :])
    l = jnp.dot(h, w3_ref[:, :], preferred_element_type=jnp.float32) \
        + b3_ref[:, :]
    mx = jnp.max(l, axis=1, keepdims=True)
    e = jnp.exp(l - mx)
    out_ref[:, :] = e / jnp.sum(e, axis=1, keepdims=True)


def _run_head(f, w1, b1, w2, b2, w3, b3):
    nc = w3.shape[1]
    return pl.pallas_call(
        _head_kernel,
        out_shape=jax.ShapeDtypeStruct((_B, nc), jnp.float32),
    )(f, w1, b1, w2, b2, w3, b3)


def kernel(input, sa1_w0, sa1_b0, sa1_w1, sa1_b1, sa1_w2, sa1_b2,
           sa2_w0, sa2_b0, sa2_w1, sa2_b1, sa2_w2, sa2_b2,
           sa3_w0, sa3_b0, sa3_w1, sa3_b1, sa3_w2, sa3_b2,
           d1_w, d1_b, d2_w, d2_b, d3_w, d3_b):
    col = lambda b: b.reshape(-1, 1)
    row = lambda b: b.reshape(1, -1)
    xt = input.transpose(2, 0, 1)
    xbt = input.transpose(0, 2, 1)
    c1x, c1y, c1z, c2x, c2y, c2z = _run_fps(xt)
    ct1 = jnp.stack([c1x, c1y, c1z], axis=1)
    ct2 = jnp.stack([c2x, c2y, c2z], axis=1)

    idxt1 = _run_bq(xbt, ct1, _N1, _M1, _NS1, _R1)
    pts1t = _run_sa(xbt, ct1, idxt1, sa1_w0.T, col(sa1_b0), sa1_w1.T,
                    col(sa1_b1), sa1_w2.T, col(sa1_b2),
                    _N1, _M1, _NS1)

    p2t = jnp.concatenate([ct1, pts1t], axis=1)
    idxt2 = _run_bq(ct1, ct2, _M1, _M2, _NS2, _R2)
    pts2t = _run_sa(p2t, ct2, idxt2, sa2_w0.T, col(sa2_b0), sa2_w1.T,
                    col(sa2_b1), sa2_w2.T, col(sa2_b2),
                    _M1, _M2, _NS2)

    f3t = jnp.concatenate([ct2, pts2t], axis=1)
    feat = _run_sa3(f3t, sa3_w0.T, col(sa3_b0), sa3_w1.T, col(sa3_b1),
                    sa3_w2.T, col(sa3_b2))
    return _run_head(feat, d1_w, row(d1_b), d2_w, row(d2_b),
                     d3_w, row(d3_b))


# speedup vs baseline: 1.3471x; 1.0306x over previous
import functools

import jax
import jax.numpy as jnp
from jax.experimental import pallas as pl

_B = 16
_N1, _M1, _NS1, _R1 = 1024, 512, 32, 0.2
_N2, _M2, _NS2, _R2 = 512, 128, 64, 0.4


def _lrelu(x):
    return jnp.where(x >= 0, x, 0.01 * x)


def _fps_body(x, y, z, n, npoint, cx_ref, cy_ref, cz_ref):
    iota = jax.lax.broadcasted_iota(jnp.int32, (_B, n), 1)
    iota_c = jax.lax.broadcasted_iota(jnp.int32, (_B, npoint), 1)

    def body(i, carry):
        dist, far, cx, cy, cz = carry
        oh = iota == far
        xf = jnp.sum(jnp.where(oh, x, 0.0), axis=1, keepdims=True)
        yf = jnp.sum(jnp.where(oh, y, 0.0), axis=1, keepdims=True)
        zf = jnp.sum(jnp.where(oh, z, 0.0), axis=1, keepdims=True)
        sel = iota_c == i
        cx = jnp.where(sel, xf, cx)
        cy = jnp.where(sel, yf, cy)
        cz = jnp.where(sel, zf, cz)
        dx = x - xf
        dy = y - yf
        dz = z - zf
        d = dx * dx + dy * dy + dz * dz
        dist = jnp.minimum(dist, d)
        m = jnp.max(dist, axis=1, keepdims=True)
        far = jnp.min(jnp.where(dist == m, iota, n), axis=1, keepdims=True)
        return dist, far, cx, cy, cz

    dist0 = jnp.full((_B, n), 1e10, dtype=jnp.float32)
    far0 = jnp.zeros((_B, 1), dtype=jnp.int32)
    c0 = jnp.zeros((_B, npoint), dtype=jnp.float32)
    _, _, cx, cy, cz = jax.lax.fori_loop(0, npoint, body,
                                         (dist0, far0, c0, c0, c0))
    cx_ref[:, :] = cx
    cy_ref[:, :] = cy
    cz_ref[:, :] = cz


def _fps_kernel(xt_ref, c1x_ref, c1y_ref, c1z_ref, c2x_ref, c2y_ref, c2z_ref):
    x = xt_ref[0]
    y = xt_ref[1]
    z = xt_ref[2]
    _fps_body(x, y, z, _N1, _M1, c1x_ref, c1y_ref, c1z_ref)
    x2 = c1x_ref[:, :]
    y2 = c1y_ref[:, :]
    z2 = c1z_ref[:, :]
    _fps_body(x2, y2, z2, _M1, _M2, c2x_ref, c2y_ref, c2z_ref)


def _run_fps(xt):
    f32 = jnp.float32
    outs = [
        jax.ShapeDtypeStruct((_B, _M1), f32), jax.ShapeDtypeStruct((_B, _M1), f32),
        jax.ShapeDtypeStruct((_B, _M1), f32), jax.ShapeDtypeStruct((_B, _M2), f32),
        jax.ShapeDtypeStruct((_B, _M2), f32), jax.ShapeDtypeStruct((_B, _M2), f32),
    ]
    return pl.pallas_call(_fps_kernel, out_shape=outs)(xt)


def _bq_kernel(pt_ref, ct_ref, idxt_ref, *, n, m, ns, r2):
    px = pt_ref[0, 0:1, :]
    py = pt_ref[0, 1:2, :]
    pz = pt_ref[0, 2:3, :]
    cx = ct_ref[0, 0:1, :].T
    cy = ct_ref[0, 1:2, :].T
    cz = ct_ref[0, 2:3, :].T
    dx = cx - px
    dy = cy - py
    dz = cz - pz
    d2 = dx * dx + dy * dy + dz * dz
    big = float(n)
    iota = jax.lax.broadcasted_iota(jnp.int32, (m, n), 1).astype(jnp.float32)
    v0 = jnp.where(d2 <= r2, iota, big)
    iota_s = jax.lax.broadcasted_iota(jnp.int32, (ns, m), 0)

    def body(k, carry):
        v, idx0, out = carry
        mn = jnp.min(v, axis=1, keepdims=True)
        sel = jnp.where(mn >= big, idx0, mn)
        idx0 = jnp.where(k == 0, sel, idx0)
        out = jnp.where(iota_s == k, sel.T, out)
        v = jnp.where(v == mn, big, v)
        return v, idx0, out

    _, _, out = jax.lax.fori_loop(
        0, ns, body,
        (v0, jnp.zeros((m, 1), jnp.float32), jnp.zeros((ns, m), jnp.float32)))
    idxt_ref[0] = out.astype(jnp.int32)


def _run_bq(pt, ct, n, m, ns, r):
    kfn = functools.partial(_bq_kernel, n=n, m=m, ns=ns, r2=r * r)
    return pl.pallas_call(
        kfn,
        grid=(_B,),
        in_specs=[
            pl.BlockSpec((1, 3, n), lambda b: (b, 0, 0)),
            pl.BlockSpec((1, 3, m), lambda b: (b, 0, 0)),
        ],
        out_specs=pl.BlockSpec((1, ns, m), lambda b: (b, 0, 0)),
        out_shape=jax.ShapeDtypeStruct((_B, ns, m), jnp.int32),
    )(pt, ct)


def _sa_kernel(pt_ref, ct_ref, idxt_ref, w0t_ref, b0_ref, w1t_ref, b1_ref,
               w2t_ref, b2_ref, out_ref, *, m, ns, nch, g_blk):
    w0t = w0t_ref[:, :]
    a_t = jnp.dot(w0t, pt_ref[0], preferred_element_type=jnp.float32)
    bc = jnp.dot(w0t[:, 0:3], ct_ref[0],
                 preferred_element_type=jnp.float32)
    c1 = w0t.shape[0]
    c2 = w2t_ref.shape[0]
    off = bc - b0_ref[:, :]
    w1t = w1t_ref[:, :]
    b1 = b1_ref[:, :]
    w2t = w2t_ref[:, :]
    b2 = b2_ref[:, :]

    def body(k, acc):
        parts = []
        for t in range(g_blk):
            idxk = idxt_ref[0, pl.ds(k * g_blk + t, 1), :]
            idxb = jnp.broadcast_to(idxk, (c1, m))
            lo = jax.lax.rem(idxb, 128)
            hi = jax.lax.div(idxb, 128)
            g = jnp.zeros((c1, m), jnp.float32)
            for ch in range(nch):
                src = a_t[:, ch * 128:(ch + 1) * 128]
                gc = jnp.take_along_axis(src, lo, axis=1,
                                         mode="promise_in_bounds")
                g = jnp.where(hi == ch, gc, g)
            parts.append(_lrelu(g - off))
        h = jnp.concatenate(parts, axis=1)
        h = _lrelu(jnp.dot(w1t, h, preferred_element_type=jnp.float32) + b1)
        h = _lrelu(jnp.dot(w2t, h, preferred_element_type=jnp.float32) + b2)
        for t in range(g_blk):
            acc = jnp.maximum(acc, h[:, t * m:(t + 1) * m])
        return acc

    out_ref[0] = jax.lax.fori_loop(
        0, ns // g_blk, body, jnp.full((c2, m), -jnp.inf, jnp.float32))


def _run_sa(pt, ct, idxt, w0t, b0, w1t, b1, w2t, b2, n, m, ns, g_blk):
    c1, cin = w0t.shape
    c2 = w2t.shape[0]
    kfn = functools.partial(_sa_kernel, m=m, ns=ns, nch=n // 128,
                            g_blk=g_blk)
    wspec = lambda s: pl.BlockSpec(s, lambda b: (0,) * len(s))
    return pl.pallas_call(
        kfn,
        grid=(_B,),
        in_specs=[
            pl.BlockSpec((1, cin, n), lambda b: (b, 0, 0)),
            pl.BlockSpec((1, 3, m), lambda b: (b, 0, 0)),
            pl.BlockSpec((1, ns, m), lambda b: (b, 0, 0)),
            wspec((c1, cin)), wspec((c1, 1)),
            wspec((c1, c1)), wspec((c1, 1)),
            wspec((c2, c1)), wspec((c2, 1)),
        ],
        out_specs=pl.BlockSpec((1, c2, m), lambda b: (b, 0, 0)),
        out_shape=jax.ShapeDtypeStruct((_B, c2, m), jnp.float32),
    )(pt, ct, idxt, w0t, b0, w1t, b1, w2t, b2)


def _sa3_kernel(ft_ref, w0t_ref, b0_ref, w1t_ref, b1_ref, w2t_ref, b2_ref,
                out_ref):
    h = _lrelu(jnp.dot(w0t_ref[:, :], ft_ref[:, :],
                       preferred_element_type=jnp.float32) + b0_ref[:, :])
    h = _lrelu(jnp.dot(w1t_ref[:, :], h,
                       preferred_element_type=jnp.float32) + b1_ref[:, :])
    h = _lrelu(jnp.dot(w2t_ref[:, :], h,
                       preferred_element_type=jnp.float32) + b2_ref[:, :])
    cols = [jnp.max(h[:, b * _M2:(b + 1) * _M2], axis=1, keepdims=True)
            for b in range(_B)]
    out_ref[:, :] = jnp.concatenate(cols, axis=1)


def _run_sa3(ft, w0t, b0, w1t, b1, w2t, b2):
    c2 = w2t.shape[0]
    ftb = ft.transpose(1, 0, 2).reshape(ft.shape[1], _B * _M2)
    return pl.pallas_call(
        _sa3_kernel,
        out_shape=jax.ShapeDtypeStruct((c2, _B), jnp.float32),
    )(ftb, w0t, b0, w1t, b1, w2t, b2).T


def _head_kernel(f_ref, w1_ref, b1_ref, w2_ref, b2_ref, w3_ref, b3_ref,
                 out_ref):
    h = _lrelu(jnp.dot(f_ref[:, :], w1_ref[:, :],
                       preferred_element_type=jnp.float32) + b1_ref[:, :])
    h = _lrelu(jnp.dot(h, w2_ref[:, :],
                       preferred_element_type=jnp.float32) + b2_ref[:, :])
    l = jnp.dot(h, w3_ref[:, :], preferred_element_type=jnp.float32) \
        + b3_ref[:, :]
    mx = jnp.max(l, axis=1, keepdims=True)
    e = jnp.exp(l - mx)
    out_ref[:, :] = e / jnp.sum(e, axis=1, keepdims=True)


def _run_head(f, w1, b1, w2, b2, w3, b3):
    nc = w3.shape[1]
    return pl.pallas_call(
        _head_kernel,
        out_shape=jax.ShapeDtypeStruct((_B, nc), jnp.float32),
    )(f, w1, b1, w2, b2, w3, b3)


def kernel(input, sa1_w0, sa1_b0, sa1_w1, sa1_b1, sa1_w2, sa1_b2,
           sa2_w0, sa2_b0, sa2_w1, sa2_b1, sa2_w2, sa2_b2,
           sa3_w0, sa3_b0, sa3_w1, sa3_b1, sa3_w2, sa3_b2,
           d1_w, d1_b, d2_w, d2_b, d3_w, d3_b):
    col = lambda b: b.reshape(-1, 1)
    row = lambda b: b.reshape(1, -1)
    xt = input.transpose(2, 0, 1)
    xbt = input.transpose(0, 2, 1)
    c1x, c1y, c1z, c2x, c2y, c2z = _run_fps(xt)
    ct1 = jnp.stack([c1x, c1y, c1z], axis=1)
    ct2 = jnp.stack([c2x, c2y, c2z], axis=1)

    idxt1 = _run_bq(xbt, ct1, _N1, _M1, _NS1, _R1)
    pts1t = _run_sa(xbt, ct1, idxt1, sa1_w0.T, col(sa1_b0), sa1_w1.T,
                    col(sa1_b1), sa1_w2.T, col(sa1_b2),
                    _N1, _M1, _NS1, 16)

    p2t = jnp.concatenate([ct1, pts1t], axis=1)
    idxt2 = _run_bq(ct1, ct2, _M1, _M2, _NS2, _R2)
    pts2t = _run_sa(p2t, ct2, idxt2, sa2_w0.T, col(sa2_b0), sa2_w1.T,
                    col(sa2_b1), sa2_w2.T, col(sa2_b2),
                    _M1, _M2, _NS2, 16)

    f3t = jnp.concatenate([ct2, pts2t], axis=1)
    feat = _run_sa3(f3t, sa3_w0.T, col(sa3_b0), sa3_w1.T, col(sa3_b1),
                    sa3_w2.T, col(sa3_b2))
    return _run_head(feat, d1_w, row(d1_b), d2_w, row(d2_b),
                     d3_w, row(d3_b))


# speedup vs baseline: 1.7453x; 1.2956x over previous
import functools

import jax
import jax.numpy as jnp
from jax.experimental import pallas as pl

_B = 16
_N1, _M1, _NS1, _R1 = 1024, 512, 32, 0.2
_N2, _M2, _NS2, _R2 = 512, 128, 64, 0.4


def _lrelu(x):
    return jnp.where(x >= 0, x, 0.01 * x)


def _fps_body(x, y, z, n, npoint, cx_ref, cy_ref, cz_ref):
    iota = jax.lax.broadcasted_iota(jnp.int32, (_B, n), 1)
    iota_c = jax.lax.broadcasted_iota(jnp.int32, (_B, npoint), 1)

    def body(i, carry):
        dist, far, cx, cy, cz = carry
        oh = iota == far
        xf = jnp.sum(jnp.where(oh, x, 0.0), axis=1, keepdims=True)
        yf = jnp.sum(jnp.where(oh, y, 0.0), axis=1, keepdims=True)
        zf = jnp.sum(jnp.where(oh, z, 0.0), axis=1, keepdims=True)
        sel = iota_c == i
        cx = jnp.where(sel, xf, cx)
        cy = jnp.where(sel, yf, cy)
        cz = jnp.where(sel, zf, cz)
        dx = x - xf
        dy = y - yf
        dz = z - zf
        d = dx * dx + dy * dy + dz * dz
        dist = jnp.minimum(dist, d)
        m = jnp.max(dist, axis=1, keepdims=True)
        far = jnp.min(jnp.where(dist == m, iota, n), axis=1, keepdims=True)
        return dist, far, cx, cy, cz

    dist0 = jnp.full((_B, n), 1e10, dtype=jnp.float32)
    far0 = jnp.zeros((_B, 1), dtype=jnp.int32)
    c0 = jnp.zeros((_B, npoint), dtype=jnp.float32)
    _, _, cx, cy, cz = jax.lax.fori_loop(0, npoint, body,
                                         (dist0, far0, c0, c0, c0))
    cx_ref[:, :] = cx
    cy_ref[:, :] = cy
    cz_ref[:, :] = cz


def _fps_kernel(xt_ref, c1x_ref, c1y_ref, c1z_ref, c2x_ref, c2y_ref, c2z_ref):
    x = xt_ref[0]
    y = xt_ref[1]
    z = xt_ref[2]
    _fps_body(x, y, z, _N1, _M1, c1x_ref, c1y_ref, c1z_ref)
    x2 = c1x_ref[:, :]
    y2 = c1y_ref[:, :]
    z2 = c1z_ref[:, :]
    _fps_body(x2, y2, z2, _M1, _M2, c2x_ref, c2y_ref, c2z_ref)


def _run_fps(xt):
    f32 = jnp.float32
    outs = [
        jax.ShapeDtypeStruct((_B, _M1), f32), jax.ShapeDtypeStruct((_B, _M1), f32),
        jax.ShapeDtypeStruct((_B, _M1), f32), jax.ShapeDtypeStruct((_B, _M2), f32),
        jax.ShapeDtypeStruct((_B, _M2), f32), jax.ShapeDtypeStruct((_B, _M2), f32),
    ]
    return pl.pallas_call(_fps_kernel, out_shape=outs)(xt)


def _bq_kernel(pt_ref, ct_ref, idxt_ref, *, n, m, ns, r2):
    px = pt_ref[0, 0:1, :]
    py = pt_ref[0, 1:2, :]
    pz = pt_ref[0, 2:3, :]
    cx = ct_ref[0, 0:1, :].T
    cy = ct_ref[0, 1:2, :].T
    cz = ct_ref[0, 2:3, :].T
    dx = cx - px
    dy = cy - py
    dz = cz - pz
    d2 = dx * dx + dy * dy + dz * dz
    big = float(n)
    iota = jax.lax.broadcasted_iota(jnp.int32, (m, n), 1).astype(jnp.float32)
    v0 = jnp.where(d2 <= r2, iota, big)
    iota_s = jax.lax.broadcasted_iota(jnp.int32, (ns, m), 0)

    def body(k, carry):
        v, idx0, out = carry
        mn = jnp.min(v, axis=1, keepdims=True)
        sel = jnp.where(mn >= big, idx0, mn)
        idx0 = jnp.where(k == 0, sel, idx0)
        out = jnp.where(iota_s == k, sel.T, out)
        v = jnp.where(v == mn, big, v)
        return v, idx0, out

    _, _, out = jax.lax.fori_loop(
        0, ns, body,
        (v0, jnp.zeros((m, 1), jnp.float32), jnp.zeros((ns, m), jnp.float32)))
    idxt_ref[0] = out.astype(jnp.int32)


def _run_bq(pt, ct, n, m, ns, r):
    kfn = functools.partial(_bq_kernel, n=n, m=m, ns=ns, r2=r * r)
    return pl.pallas_call(
        kfn,
        grid=(_B,),
        in_specs=[
            pl.BlockSpec((1, 3, n), lambda b: (b, 0, 0)),
            pl.BlockSpec((1, 3, m), lambda b: (b, 0, 0)),
        ],
        out_specs=pl.BlockSpec((1, ns, m), lambda b: (b, 0, 0)),
        out_shape=jax.ShapeDtypeStruct((_B, ns, m), jnp.int32),
    )(pt, ct)


def _sa1_kernel(pt_ref, ct_ref, idxt_ref, w0t_ref, b0_ref, w1t_ref, b1_ref,
                w2t_ref, b2_ref, out_ref, *, n, m, ns, nch, g_blk):
    x8 = jnp.concatenate(
        [pt_ref[0], jnp.zeros((5, n), jnp.float32)], axis=0)
    w0t = w0t_ref[:, :]
    bc = jnp.dot(w0t, ct_ref[0], preferred_element_type=jnp.float32)
    c1 = w0t.shape[0]
    c2 = w2t_ref.shape[0]
    off = bc - b0_ref[:, :]
    off_t = jnp.concatenate([off] * g_blk, axis=1)
    w1t = w1t_ref[:, :]
    b1 = b1_ref[:, :]
    w2t = w2t_ref[:, :]
    b2 = b2_ref[:, :]

    def body(k, acc):
        parts = []
        for t in range(g_blk):
            idxk = idxt_ref[0, pl.ds(k * g_blk + t, 1), :]
            idxb = jnp.broadcast_to(idxk, (8, m))
            lo = jax.lax.rem(idxb, 128)
            hi = jax.lax.div(idxb, 128)
            g = jnp.zeros((8, m), jnp.float32)
            for ch in range(nch):
                src = x8[:, ch * 128:(ch + 1) * 128]
                gc = jnp.take_along_axis(src, lo, axis=1,
                                         mode="promise_in_bounds")
                g = jnp.where(hi == ch, gc, g)
            parts.append(g)
        gx = jnp.concatenate(parts, axis=1)
        h = _lrelu(jnp.dot(w0t, gx[0:3, :],
                           preferred_element_type=jnp.float32) - off_t)
        h = _lrelu(jnp.dot(w1t, h, preferred_element_type=jnp.float32) + b1)
        h = _lrelu(jnp.dot(w2t, h, preferred_element_type=jnp.float32) + b2)
        for t in range(g_blk):
            acc = jnp.maximum(acc, h[:, t * m:(t + 1) * m])
        return acc

    out_ref[0] = jax.lax.fori_loop(
        0, ns // g_blk, body, jnp.full((c2, m), -jnp.inf, jnp.float32))


def _run_sa1(pt, ct, idxt, w0t, b0, w1t, b1, w2t, b2, n, m, ns, g_blk):
    c1 = w0t.shape[0]
    c2 = w2t.shape[0]
    kfn = functools.partial(_sa1_kernel, n=n, m=m, ns=ns, nch=n // 128,
                            g_blk=g_blk)
    wspec = lambda s: pl.BlockSpec(s, lambda b: (0,) * len(s))
    return pl.pallas_call(
        kfn,
        grid=(_B,),
        in_specs=[
            pl.BlockSpec((1, 3, n), lambda b: (b, 0, 0)),
            pl.BlockSpec((1, 3, m), lambda b: (b, 0, 0)),
            pl.BlockSpec((1, ns, m), lambda b: (b, 0, 0)),
            wspec((c1, 3)), wspec((c1, 1)),
            wspec((c1, c1)), wspec((c1, 1)),
            wspec((c2, c1)), wspec((c2, 1)),
        ],
        out_specs=pl.BlockSpec((1, c2, m), lambda b: (b, 0, 0)),
        out_shape=jax.ShapeDtypeStruct((_B, c2, m), jnp.float32),
    )(pt, ct, idxt, w0t, b0, w1t, b1, w2t, b2)


def _sa_kernel(pt_ref, ct_ref, idxt_ref, w0t_ref, b0_ref, w1t_ref, b1_ref,
               w2t_ref, b2_ref, out_ref, *, m, ns, nch, g_blk):
    w0t = w0t_ref[:, :]
    a_t = jnp.dot(w0t, pt_ref[0], preferred_element_type=jnp.float32)
    bc = jnp.dot(w0t[:, 0:3], ct_ref[0],
                 preferred_element_type=jnp.float32)
    c1 = w0t.shape[0]
    c2 = w2t_ref.shape[0]
    off = bc - b0_ref[:, :]
    w1t = w1t_ref[:, :]
    b1 = b1_ref[:, :]
    w2t = w2t_ref[:, :]
    b2 = b2_ref[:, :]

    def body(k, acc):
        parts = []
        for t in range(g_blk):
            idxk = idxt_ref[0, pl.ds(k * g_blk + t, 1), :]
            idxb = jnp.broadcast_to(idxk, (c1, m))
            lo = jax.lax.rem(idxb, 128)
            hi = jax.lax.div(idxb, 128)
            g = jnp.zeros((c1, m), jnp.float32)
            for ch in range(nch):
                src = a_t[:, ch * 128:(ch + 1) * 128]
                gc = jnp.take_along_axis(src, lo, axis=1,
                                         mode="promise_in_bounds")
                g = jnp.where(hi == ch, gc, g)
            parts.append(_lrelu(g - off))
        h = jnp.concatenate(parts, axis=1)
        h = _lrelu(jnp.dot(w1t, h, preferred_element_type=jnp.float32) + b1)
        h = _lrelu(jnp.dot(w2t, h, preferred_element_type=jnp.float32) + b2)
        for t in range(g_blk):
            acc = jnp.maximum(acc, h[:, t * m:(t + 1) * m])
        return acc

    out_ref[0] = jax.lax.fori_loop(
        0, ns // g_blk, body, jnp.full((c2, m), -jnp.inf, jnp.float32))


def _run_sa(pt, ct, idxt, w0t, b0, w1t, b1, w2t, b2, n, m, ns, g_blk):
    c1, cin = w0t.shape
    c2 = w2t.shape[0]
    kfn = functools.partial(_sa_kernel, m=m, ns=ns, nch=n // 128,
                            g_blk=g_blk)
    wspec = lambda s: pl.BlockSpec(s, lambda b: (0,) * len(s))
    return pl.pallas_call(
        kfn,
        grid=(_B,),
        in_specs=[
            pl.BlockSpec((1, cin, n), lambda b: (b, 0, 0)),
            pl.BlockSpec((1, 3, m), lambda b: (b, 0, 0)),
            pl.BlockSpec((1, ns, m), lambda b: (b, 0, 0)),
            wspec((c1, cin)), wspec((c1, 1)),
            wspec((c1, c1)), wspec((c1, 1)),
            wspec((c2, c1)), wspec((c2, 1)),
        ],
        out_specs=pl.BlockSpec((1, c2, m), lambda b: (b, 0, 0)),
        out_shape=jax.ShapeDtypeStruct((_B, c2, m), jnp.float32),
    )(pt, ct, idxt, w0t, b0, w1t, b1, w2t, b2)


def _sa3_kernel(ft_ref, w0t_ref, b0_ref, w1t_ref, b1_ref, w2t_ref, b2_ref,
                out_ref):
    h = _lrelu(jnp.dot(w0t_ref[:, :], ft_ref[:, :],
                       preferred_element_type=jnp.float32) + b0_ref[:, :])
    h = _lrelu(jnp.dot(w1t_ref[:, :], h,
                       preferred_element_type=jnp.float32) + b1_ref[:, :])
    h = _lrelu(jnp.dot(w2t_ref[:, :], h,
                       preferred_element_type=jnp.float32) + b2_ref[:, :])
    cols = [jnp.max(h[:, b * _M2:(b + 1) * _M2], axis=1, keepdims=True)
            for b in range(_B)]
    out_ref[:, :] = jnp.concatenate(cols, axis=1)


def _run_sa3(ft, w0t, b0, w1t, b1, w2t, b2):
    c2 = w2t.shape[0]
    ftb = ft.transpose(1, 0, 2).reshape(ft.shape[1], _B * _M2)
    return pl.pallas_call(
        _sa3_kernel,
        out_shape=jax.ShapeDtypeStruct((c2, _B), jnp.float32),
    )(ftb, w0t, b0, w1t, b1, w2t, b2).T


def _head_kernel(f_ref, w1_ref, b1_ref, w2_ref, b2_ref, w3_ref, b3_ref,
                 out_ref):
    h = _lrelu(jnp.dot(f_ref[:, :], w1_ref[:, :],
                       preferred_element_type=jnp.float32) + b1_ref[:, :])
    h = _lrelu(jnp.dot(h, w2_ref[:, :],
                       preferred_element_type=jnp.float32) + b2_ref[:, :])
    l = jnp.dot(h, w3_ref[:, :], preferred_element_type=jnp.float32) \
        + b3_ref[:, :]
    mx = jnp.max(l, axis=1, keepdims=True)
    e = jnp.exp(l - mx)
    out_ref[:, :] = e / jnp.sum(e, axis=1, keepdims=True)


def _run_head(f, w1, b1, w2, b2, w3, b3):
    nc = w3.shape[1]
    return pl.pallas_call(
        _head_kernel,
        out_shape=jax.ShapeDtypeStruct((_B, nc), jnp.float32),
    )(f, w1, b1, w2, b2, w3, b3)


def kernel(input, sa1_w0, sa1_b0, sa1_w1, sa1_b1, sa1_w2, sa1_b2,
           sa2_w0, sa2_b0, sa2_w1, sa2_b1, sa2_w2, sa2_b2,
           sa3_w0, sa3_b0, sa3_w1, sa3_b1, sa3_w2, sa3_b2,
           d1_w, d1_b, d2_w, d2_b, d3_w, d3_b):
    col = lambda b: b.reshape(-1, 1)
    row = lambda b: b.reshape(1, -1)
    xt = input.transpose(2, 0, 1)
    xbt = input.transpose(0, 2, 1)
    c1x, c1y, c1z, c2x, c2y, c2z = _run_fps(xt)
    ct1 = jnp.stack([c1x, c1y, c1z], axis=1)
    ct2 = jnp.stack([c2x, c2y, c2z], axis=1)

    idxt1 = _run_bq(xbt, ct1, _N1, _M1, _NS1, _R1)
    pts1t = _run_sa1(xbt, ct1, idxt1, sa1_w0.T, col(sa1_b0), sa1_w1.T,
                     col(sa1_b1), sa1_w2.T, col(sa1_b2),
                     _N1, _M1, _NS1, 16)

    p2t = jnp.concatenate([ct1, pts1t], axis=1)
    idxt2 = _run_bq(ct1, ct2, _M1, _M2, _NS2, _R2)
    pts2t = _run_sa(p2t, ct2, idxt2, sa2_w0.T, col(sa2_b0), sa2_w1.T,
                    col(sa2_b1), sa2_w2.T, col(sa2_b2),
                    _M1, _M2, _NS2, 16)

    f3t = jnp.concatenate([ct2, pts2t], axis=1)
    feat = _run_sa3(f3t, sa3_w0.T, col(sa3_b0), sa3_w1.T, col(sa3_b1),
                    sa3_w2.T, col(sa3_b2))
    return _run_head(feat, d1_w, row(d1_b), d2_w, row(d2_b),
                     d3_w, row(d3_b))


# speedup vs baseline: 2.3190x; 1.3287x over previous
import functools

import jax
import jax.numpy as jnp
from jax.experimental import pallas as pl

_B = 16
_N1, _M1, _NS1, _R1 = 1024, 512, 32, 0.2
_N2, _M2, _NS2, _R2 = 512, 128, 64, 0.4


def _lrelu(x):
    return jnp.where(x >= 0, x, 0.01 * x)


def _fps_body(x, y, z, n, npoint, cx_ref, cy_ref, cz_ref):
    iota = jax.lax.broadcasted_iota(jnp.int32, (_B, n), 1)
    iota_c = jax.lax.broadcasted_iota(jnp.int32, (_B, npoint), 1)

    def body(i, carry):
        dist, far, cx, cy, cz = carry
        oh = iota == far
        xf = jnp.sum(jnp.where(oh, x, 0.0), axis=1, keepdims=True)
        yf = jnp.sum(jnp.where(oh, y, 0.0), axis=1, keepdims=True)
        zf = jnp.sum(jnp.where(oh, z, 0.0), axis=1, keepdims=True)
        sel = iota_c == i
        cx = jnp.where(sel, xf, cx)
        cy = jnp.where(sel, yf, cy)
        cz = jnp.where(sel, zf, cz)
        dx = x - xf
        dy = y - yf
        dz = z - zf
        d = dx * dx + dy * dy + dz * dz
        dist = jnp.minimum(dist, d)
        m = jnp.max(dist, axis=1, keepdims=True)
        far = jnp.min(jnp.where(dist == m, iota, n), axis=1, keepdims=True)
        return dist, far, cx, cy, cz

    dist0 = jnp.full((_B, n), 1e10, dtype=jnp.float32)
    far0 = jnp.zeros((_B, 1), dtype=jnp.int32)
    c0 = jnp.zeros((_B, npoint), dtype=jnp.float32)
    _, _, cx, cy, cz = jax.lax.fori_loop(0, npoint, body,
                                         (dist0, far0, c0, c0, c0))
    cx_ref[:, :] = cx
    cy_ref[:, :] = cy
    cz_ref[:, :] = cz


def _fps_kernel(xt_ref, c1x_ref, c1y_ref, c1z_ref, c2x_ref, c2y_ref, c2z_ref):
    x = xt_ref[0]
    y = xt_ref[1]
    z = xt_ref[2]
    _fps_body(x, y, z, _N1, _M1, c1x_ref, c1y_ref, c1z_ref)
    x2 = c1x_ref[:, :]
    y2 = c1y_ref[:, :]
    z2 = c1z_ref[:, :]
    _fps_body(x2, y2, z2, _M1, _M2, c2x_ref, c2y_ref, c2z_ref)


def _run_fps(xt):
    f32 = jnp.float32
    outs = [
        jax.ShapeDtypeStruct((_B, _M1), f32), jax.ShapeDtypeStruct((_B, _M1), f32),
        jax.ShapeDtypeStruct((_B, _M1), f32), jax.ShapeDtypeStruct((_B, _M2), f32),
        jax.ShapeDtypeStruct((_B, _M2), f32), jax.ShapeDtypeStruct((_B, _M2), f32),
    ]
    return pl.pallas_call(_fps_kernel, out_shape=outs)(xt)


def _bq_kernel(pt_ref, ct_ref, lt_ref, idxt_ref, *, n, m, ns, r2):
    px = pt_ref[0, 0:1, :]
    py = pt_ref[0, 1:2, :]
    pz = pt_ref[0, 2:3, :]
    cx = ct_ref[0, 0:1, :].T
    cy = ct_ref[0, 1:2, :].T
    cz = ct_ref[0, 2:3, :].T
    dx = cx - px
    dy = cy - py
    dz = cz - pz
    d2 = dx * dx + dy * dy + dz * dz
    mask = d2 <= r2
    rank = jnp.dot(jnp.where(mask, 1.0, 0.0).astype(jnp.bfloat16),
                   lt_ref[:, :], preferred_element_type=jnp.float32)
    iota = jax.lax.broadcasted_iota(jnp.int32, (m, n), 1).astype(jnp.float32)
    jm = jnp.where(mask, iota + 1.0, 0.0)
    odd = jax.lax.rem(rank, 2.0) == 1.0
    jms = jnp.where(odd, jm * 2048.0, jm)
    rhalf = jnp.floor(rank * 0.5)
    iota_s = jax.lax.broadcasted_iota(jnp.int32, (ns, m), 0)

    def body(k, carry):
        fv, out = carry
        s = jnp.sum(jnp.where(rhalf == k.astype(jnp.float32), jms, 0.0),
                    axis=1, keepdims=True)
        b = jnp.floor(s * (1.0 / 2048.0))
        a = s - 2048.0 * b
        a = a - 1.0
        b = b - 1.0
        fv = jnp.where(k == 0, a, fv)
        va = jnp.where(a < 0, fv, a)
        vb = jnp.where(b < 0, fv, b)
        out = jnp.where(iota_s == 2 * k, va.T, out)
        out = jnp.where(iota_s == 2 * k + 1, vb.T, out)
        return fv, out

    _, out = jax.lax.fori_loop(
        0, ns // 2, body,
        (jnp.zeros((m, 1), jnp.float32), jnp.zeros((ns, m), jnp.float32)))
    idxt_ref[0] = out.astype(jnp.int32)


def _run_bq(pt, ct, n, m, ns, r):
    iot = jnp.arange(n)
    lt = (iot[:, None] < iot[None, :]).astype(jnp.bfloat16)
    kfn = functools.partial(_bq_kernel, n=n, m=m, ns=ns, r2=r * r)
    return pl.pallas_call(
        kfn,
        grid=(_B,),
        in_specs=[
            pl.BlockSpec((1, 3, n), lambda b: (b, 0, 0)),
            pl.BlockSpec((1, 3, m), lambda b: (b, 0, 0)),
            pl.BlockSpec((n, n), lambda b: (0, 0)),
        ],
        out_specs=pl.BlockSpec((1, ns, m), lambda b: (b, 0, 0)),
        out_shape=jax.ShapeDtypeStruct((_B, ns, m), jnp.int32),
    )(pt, ct, lt)


def _sa1_kernel(pt_ref, ct_ref, idxt_ref, w0t_ref, b0_ref, w1t_ref, b1_ref,
                w2t_ref, b2_ref, out_ref, *, n, m, ns, nch, g_blk):
    x8 = jnp.concatenate(
        [pt_ref[0], jnp.zeros((5, n), jnp.float32)], axis=0)
    w0t = w0t_ref[:, :]
    bc = jnp.dot(w0t, ct_ref[0], preferred_element_type=jnp.float32)
    c1 = w0t.shape[0]
    c2 = w2t_ref.shape[0]
    off = bc - b0_ref[:, :]
    off_t = jnp.concatenate([off] * g_blk, axis=1)
    w1t = w1t_ref[:, :]
    b1 = b1_ref[:, :]
    w2t = w2t_ref[:, :]
    b2 = b2_ref[:, :]

    def body(k, acc):
        parts = []
        for t in range(g_blk):
            idxk = idxt_ref[0, pl.ds(k * g_blk + t, 1), :]
            idxb = jnp.broadcast_to(idxk, (8, m))
            lo = jax.lax.rem(idxb, 128)
            hi = jax.lax.div(idxb, 128)
            g = jnp.zeros((8, m), jnp.float32)
            for ch in range(nch):
                src = x8[:, ch * 128:(ch + 1) * 128]
                gc = jnp.take_along_axis(src, lo, axis=1,
                                         mode="promise_in_bounds")
                g = jnp.where(hi == ch, gc, g)
            parts.append(g)
        gx = jnp.concatenate(parts, axis=1)
        h = _lrelu(jnp.dot(w0t, gx[0:3, :],
                           preferred_element_type=jnp.float32) - off_t)
        h = _lrelu(jnp.dot(w1t, h, preferred_element_type=jnp.float32) + b1)
        h = _lrelu(jnp.dot(w2t, h, preferred_element_type=jnp.float32) + b2)
        for t in range(g_blk):
            acc = jnp.maximum(acc, h[:, t * m:(t + 1) * m])
        return acc

    out_ref[0] = jax.lax.fori_loop(
        0, ns // g_blk, body, jnp.full((c2, m), -jnp.inf, jnp.float32))


def _run_sa1(pt, ct, idxt, w0t, b0, w1t, b1, w2t, b2, n, m, ns, g_blk):
    c1 = w0t.shape[0]
    c2 = w2t.shape[0]
    kfn = functools.partial(_sa1_kernel, n=n, m=m, ns=ns, nch=n // 128,
                            g_blk=g_blk)
    wspec = lambda s: pl.BlockSpec(s, lambda b: (0,) * len(s))
    return pl.pallas_call(
        kfn,
        grid=(_B,),
        in_specs=[
            pl.BlockSpec((1, 3, n), lambda b: (b, 0, 0)),
            pl.BlockSpec((1, 3, m), lambda b: (b, 0, 0)),
            pl.BlockSpec((1, ns, m), lambda b: (b, 0, 0)),
            wspec((c1, 3)), wspec((c1, 1)),
            wspec((c1, c1)), wspec((c1, 1)),
            wspec((c2, c1)), wspec((c2, 1)),
        ],
        out_specs=pl.BlockSpec((1, c2, m), lambda b: (b, 0, 0)),
        out_shape=jax.ShapeDtypeStruct((_B, c2, m), jnp.float32),
    )(pt, ct, idxt, w0t, b0, w1t, b1, w2t, b2)


def _sa_kernel(pt_ref, ct_ref, idxt_ref, w0t_ref, b0_ref, w1t_ref, b1_ref,
               w2t_ref, b2_ref, out_ref, *, m, ns, nch, g_blk):
    w0t = w0t_ref[:, :]
    a_t = jnp.dot(w0t, pt_ref[0], preferred_element_type=jnp.float32)
    bc = jnp.dot(w0t[:, 0:3], ct_ref[0],
                 preferred_element_type=jnp.float32)
    c1 = w0t.shape[0]
    c2 = w2t_ref.shape[0]
    off = bc - b0_ref[:, :]
    w1t = w1t_ref[:, :]
    b1 = b1_ref[:, :]
    w2t = w2t_ref[:, :]
    b2 = b2_ref[:, :]

    def body(k, acc):
        parts = []
        for t in range(g_blk):
            idxk = idxt_ref[0, pl.ds(k * g_blk + t, 1), :]
            idxb = jnp.broadcast_to(idxk, (c1, m))
            lo = jax.lax.rem(idxb, 128)
            hi = jax.lax.div(idxb, 128)
            g = jnp.zeros((c1, m), jnp.float32)
            for ch in range(nch):
                src = a_t[:, ch * 128:(ch + 1) * 128]
                gc = jnp.take_along_axis(src, lo, axis=1,
                                         mode="promise_in_bounds")
                g = jnp.where(hi == ch, gc, g)
            parts.append(_lrelu(g - off))
        h = jnp.concatenate(parts, axis=1)
        h = _lrelu(jnp.dot(w1t, h, preferred_element_type=jnp.float32) + b1)
        h = _lrelu(jnp.dot(w2t, h, preferred_element_type=jnp.float32) + b2)
        for t in range(g_blk):
            acc = jnp.maximum(acc, h[:, t * m:(t + 1) * m])
        return acc

    out_ref[0] = jax.lax.fori_loop(
        0, ns // g_blk, body, jnp.full((c2, m), -jnp.inf, jnp.float32))


def _run_sa(pt, ct, idxt, w0t, b0, w1t, b1, w2t, b2, n, m, ns, g_blk):
    c1, cin = w0t.shape
    c2 = w2t.shape[0]
    kfn = functools.partial(_sa_kernel, m=m, ns=ns, nch=n // 128,
                            g_blk=g_blk)
    wspec = lambda s: pl.BlockSpec(s, lambda b: (0,) * len(s))
    return pl.pallas_call(
        kfn,
        grid=(_B,),
        in_specs=[
            pl.BlockSpec((1, cin, n), lambda b: (b, 0, 0)),
            pl.BlockSpec((1, 3, m), lambda b: (b, 0, 0)),
            pl.BlockSpec((1, ns, m), lambda b: (b, 0, 0)),
            wspec((c1, cin)), wspec((c1, 1)),
            wspec((c1, c1)), wspec((c1, 1)),
            wspec((c2, c1)), wspec((c2, 1)),
        ],
        out_specs=pl.BlockSpec((1, c2, m), lambda b: (b, 0, 0)),
        out_shape=jax.ShapeDtypeStruct((_B, c2, m), jnp.float32),
    )(pt, ct, idxt, w0t, b0, w1t, b1, w2t, b2)


def _sa3_kernel(ft_ref, w0t_ref, b0_ref, w1t_ref, b1_ref, w2t_ref, b2_ref,
                out_ref):
    h = _lrelu(jnp.dot(w0t_ref[:, :], ft_ref[:, :],
                       preferred_element_type=jnp.float32) + b0_ref[:, :])
    h = _lrelu(jnp.dot(w1t_ref[:, :], h,
                       preferred_element_type=jnp.float32) + b1_ref[:, :])
    h = _lrelu(jnp.dot(w2t_ref[:, :], h,
                       preferred_element_type=jnp.float32) + b2_ref[:, :])
    cols = [jnp.max(h[:, b * _M2:(b + 1) * _M2], axis=1, keepdims=True)
            for b in range(_B)]
    out_ref[:, :] = jnp.concatenate(cols, axis=1)


def _run_sa3(ft, w0t, b0, w1t, b1, w2t, b2):
    c2 = w2t.shape[0]
    ftb = ft.transpose(1, 0, 2).reshape(ft.shape[1], _B * _M2)
    return pl.pallas_call(
        _sa3_kernel,
        out_shape=jax.ShapeDtypeStruct((c2, _B), jnp.float32),
    )(ftb, w0t, b0, w1t, b1, w2t, b2).T


def _head_kernel(f_ref, w1_ref, b1_ref, w2_ref, b2_ref, w3_ref, b3_ref,
                 out_ref):
    h = _lrelu(jnp.dot(f_ref[:, :], w1_ref[:, :],
                       preferred_element_type=jnp.float32) + b1_ref[:, :])
    h = _lrelu(jnp.dot(h, w2_ref[:, :],
                       preferred_element_type=jnp.float32) + b2_ref[:, :])
    l = jnp.dot(h, w3_ref[:, :], preferred_element_type=jnp.float32) \
        + b3_ref[:, :]
    mx = jnp.max(l, axis=1, keepdims=True)
    e = jnp.exp(l - mx)
    out_ref[:, :] = e / jnp.sum(e, axis=1, keepdims=True)


def _run_head(f, w1, b1, w2, b2, w3, b3):
    nc = w3.shape[1]
    return pl.pallas_call(
        _head_kernel,
        out_shape=jax.ShapeDtypeStruct((_B, nc), jnp.float32),
    )(f, w1, b1, w2, b2, w3, b3)


def kernel(input, sa1_w0, sa1_b0, sa1_w1, sa1_b1, sa1_w2, sa1_b2,
           sa2_w0, sa2_b0, sa2_w1, sa2_b1, sa2_w2, sa2_b2,
           sa3_w0, sa3_b0, sa3_w1, sa3_b1, sa3_w2, sa3_b2,
           d1_w, d1_b, d2_w, d2_b, d3_w, d3_b):
    col = lambda b: b.reshape(-1, 1)
    row = lambda b: b.reshape(1, -1)
    xt = input.transpose(2, 0, 1)
    xbt = input.transpose(0, 2, 1)
    c1x, c1y, c1z, c2x, c2y, c2z = _run_fps(xt)
    ct1 = jnp.stack([c1x, c1y, c1z], axis=1)
    ct2 = jnp.stack([c2x, c2y, c2z], axis=1)

    idxt1 = _run_bq(xbt, ct1, _N1, _M1, _NS1, _R1)
    pts1t = _run_sa1(xbt, ct1, idxt1, sa1_w0.T, col(sa1_b0), sa1_w1.T,
                     col(sa1_b1), sa1_w2.T, col(sa1_b2),
                     _N1, _M1, _NS1, 16)

    p2t = jnp.concatenate([ct1, pts1t], axis=1)
    idxt2 = _run_bq(ct1, ct2, _M1, _M2, _NS2, _R2)
    pts2t = _run_sa(p2t, ct2, idxt2, sa2_w0.T, col(sa2_b0), sa2_w1.T,
                    col(sa2_b1), sa2_w2.T, col(sa2_b2),
                    _M1, _M2, _NS2, 16)

    f3t = jnp.concatenate([ct2, pts2t], axis=1)
    feat = _run_sa3(f3t, sa3_w0.T, col(sa3_b0), sa3_w1.T, col(sa3_b1),
                    sa3_w2.T, col(sa3_b2))
    return _run_head(feat, d1_w, row(d1_b), d2_w, row(d2_b),
                     d3_w, row(d3_b))
